# baseline (speedup 1.0000x reference)
import functools

import jax
import jax.numpy as jnp
from jax import lax
from jax.experimental import pallas as pl
from jax.experimental.pallas import tpu as pltpu
from jax.experimental.pallas import tpu_sc as plsc

EPS = 1e-7
C = 81
CP = 128
BN = 2000
NW = 32
PER_W = 512


def _stats_body(ts_ref, os_ref, od_ref, td_ref, gd_ref, cnt_ref, lsum_ref):
    ts = ts_ref[0]
    osc = os_ref[0]
    iosc = lax.broadcasted_iota(jnp.int32, (BN, C), 1).astype(jnp.float32)
    label_f = jnp.sum(ts * iosc, axis=1, keepdims=True)
    rowsum = jnp.sum(osc, axis=1, keepdims=True)
    osat = jnp.sum(ts * osc, axis=1, keepdims=True)
    ratio = jnp.clip(osat / rowsum, EPS, 1.0 - EPS)
    nll = -jnp.log(ratio)

    q_div = lax.broadcasted_iota(jnp.int32, (C, 4 * C), 1) // 4
    c_row = lax.broadcasted_iota(jnp.int32, (C, 4 * C), 0)
    rmat = (q_div == c_row).astype(jnp.float32)
    q_mod = lax.broadcasted_iota(jnp.int32, (4 * C, 4), 0) % 4
    j_col = lax.broadcasted_iota(jnp.int32, (4 * C, 4), 1)
    smat = (q_mod == j_col).astype(jnp.float32)
    tsr = jnp.dot(ts, rmat, preferred_element_type=jnp.float32)
    god = jnp.dot(od_ref[0] * tsr, smat,
                  preferred_element_type=jnp.float32)
    gtd = jnp.dot(td_ref[0] * tsr, smat,
                  preferred_element_type=jnp.float32)
    gd_ref[...] = jnp.concatenate(
        [god, gtd, label_f, jnp.zeros((BN, 7), jnp.float32)], axis=1)

    pad = jnp.zeros((1, CP - C), jnp.float32)
    cvec = jnp.concatenate([jnp.sum(ts, axis=0, keepdims=True), pad], axis=1)
    lvec = jnp.concatenate([jnp.sum(ts * nll, axis=0, keepdims=True), pad],
                           axis=1)

    step = pl.program_id(0)

    @pl.when(step == 0)
    def _():
        cnt_ref[...] = jnp.zeros((1, CP), jnp.float32)
        lsum_ref[...] = jnp.zeros((1, CP), jnp.float32)

    cnt_ref[...] += cvec
    lsum_ref[...] += lvec


def _make_sc_kernel():
    mesh = plsc.VectorSubcoreMesh(core_axis_name="c", subcore_axis_name="s")

    @functools.partial(
        pl.kernel,
        mesh=mesh,
        out_type=jax.ShapeDtypeStruct((NW, 16), jnp.float32),
        compiler_params=pltpu.CompilerParams(
            needs_layout_passes=False, use_tc_tiling_on_sc=False),
        scratch_types=[
            pltpu.VMEM((PER_W, 16), jnp.float32),
            pltpu.VMEM((1, CP), jnp.float32),
            pltpu.VMEM((CP,), jnp.float32),
            pltpu.VMEM((PER_W,), jnp.float32),
            pltpu.VMEM((16,), jnp.float32),
        ],
    )
    def sc_kernel(gd_hbm, cnt_hbm, out_hbm, gd_v, cnt_v, wtab_v, w_all, acc_v):
        wid = lax.axis_index("c") * 16 + lax.axis_index("s")
        pltpu.sync_copy(gd_hbm.at[pl.ds(wid * PER_W, PER_W)], gd_v)
        pltpu.sync_copy(cnt_hbm, cnt_v)

        total = jnp.zeros((16,), jnp.float32)
        for k in range(CP // 16):
            total = total + cnt_v[0, pl.ds(k * 16, 16)]
        total = jnp.sum(total)
        p_fg = total - cnt_v[0, pl.ds(0, 16)][0]
        lane = lax.iota(jnp.int32, 16)
        for k in range(CP // 16):
            cv = cnt_v[0, pl.ds(k * 16, 16)]
            w = 1.0 / (1.0 + jnp.exp(-(p_fg / jnp.maximum(cv, EPS))))
            if k == 0:
                w = jnp.where(lane == 0, 0.0, w)
            wtab_v[pl.ds(k * 16, 16)] = w

        lane8 = jnp.full((16,), 8, jnp.int32)
        for k in range(PER_W // 16):
            lblf = plsc.load_gather(gd_v, [k * 16 + lane, lane8])
            lbl = lblf.astype(jnp.int32)
            w_all[pl.ds(k * 16, 16)] = plsc.load_gather(wtab_v, [lbl])

        def body(g, acc):
            e = g * 16 + lane
            row = e // 4
            jj = e % 4
            od = plsc.load_gather(gd_v, [row, jj])
            td = plsc.load_gather(gd_v, [row, jj + 4])
            w = plsc.load_gather(w_all, [row])
            d = jnp.abs(od - td) * w
            h = jnp.where(d < 1.0, 0.5 * d * d, d - 0.5)
            return acc + h

        acc = lax.fori_loop(0, PER_W * 4 // 16, body,
                            jnp.zeros((16,), jnp.float32))
        acc_v[...] = acc
        pltpu.sync_copy(acc_v, out_hbm.at[wid])

    return sc_kernel


def _final_body(nrows, cnt_ref, lsum_ref, part_ref, out_ref):
    cnt = cnt_ref[...]
    lsum = lsum_ref[...]
    part = part_ref[...]
    total = jnp.sum(cnt)
    p_fg = total - cnt[0, 0]
    w_cls = 1.0 / (1.0 + jnp.exp(-(total / jnp.maximum(cnt, EPS))))
    cls = jnp.sum(w_cls * lsum) / nrows
    reg = jnp.sum(part) / jnp.maximum(EPS, p_fg)
    out_ref[...] = jnp.broadcast_to(cls + reg, (1, 1))


def kernel(target_deltas, target_scores, output_deltas, output_scores):
    b, n, c = target_scores.shape
    nt = b * n

    gd_p, counts, lsum = pl.pallas_call(
        _stats_body,
        grid=(nt // BN,),
        in_specs=[
            pl.BlockSpec((b, BN, c), lambda i: (0, i, 0)),
            pl.BlockSpec((b, BN, c), lambda i: (0, i, 0)),
            pl.BlockSpec((b, BN, 4 * c), lambda i: (0, i, 0)),
            pl.BlockSpec((b, BN, 4 * c), lambda i: (0, i, 0)),
        ],
        out_specs=[
            pl.BlockSpec((BN, 16), lambda i: (i, 0)),
            pl.BlockSpec((1, CP), lambda i: (0, 0)),
            pl.BlockSpec((1, CP), lambda i: (0, 0)),
        ],
        out_shape=[
            jax.ShapeDtypeStruct((nt, 16), jnp.float32),
            jax.ShapeDtypeStruct((1, CP), jnp.float32),
            jax.ShapeDtypeStruct((1, CP), jnp.float32),
        ],
    )(target_scores, output_scores, output_deltas, target_deltas)

    npad = NW * PER_W
    gd_pad = jnp.pad(gd_p, ((0, npad - nt), (0, 0)))

    partials = _make_sc_kernel()(gd_pad, counts)

    out = pl.pallas_call(
        functools.partial(_final_body, float(nt)),
        out_shape=jax.ShapeDtypeStruct((1, 1), jnp.float32),
    )(counts, lsum, partials)
    return out[0, 0]


# speedup vs baseline: 1.6364x; 1.6364x over previous
import functools

import jax
import jax.numpy as jnp
from jax import lax
from jax.experimental import pallas as pl
from jax.experimental.pallas import tpu as pltpu
from jax.experimental.pallas import tpu_sc as plsc

EPS = 1e-7
C = 81
CP = 128
BN = 2000
NW = 32
PER_W = 512


def _stats_body(ts_ref, os_ref, od_ref, td_ref, gd_ref, cnt_ref, lsum_ref):
    ts = ts_ref[...]
    osc = os_ref[...]
    iosc = lax.broadcasted_iota(jnp.int32, (BN, C), 1).astype(jnp.float32)
    label_f = jnp.sum(ts * iosc, axis=1, keepdims=True)
    rowsum = jnp.sum(osc, axis=1, keepdims=True)
    osat = jnp.sum(ts * osc, axis=1, keepdims=True)
    ratio = jnp.clip(osat / rowsum, EPS, 1.0 - EPS)
    nll = -jnp.log(ratio)

    q_div = lax.broadcasted_iota(jnp.int32, (C, 4 * C), 1) // 4
    c_row = lax.broadcasted_iota(jnp.int32, (C, 4 * C), 0)
    rmat = (q_div == c_row).astype(jnp.float32)
    q_mod = lax.broadcasted_iota(jnp.int32, (4 * C, 4), 0) % 4
    j_col = lax.broadcasted_iota(jnp.int32, (4 * C, 4), 1)
    smat = (q_mod == j_col).astype(jnp.float32)
    tsr = jnp.dot(ts, rmat, preferred_element_type=jnp.float32)
    god = jnp.dot(od_ref[...] * tsr, smat,
                  preferred_element_type=jnp.float32)
    gtd = jnp.dot(td_ref[...] * tsr, smat,
                  preferred_element_type=jnp.float32)
    gd_ref[...] = jnp.concatenate(
        [god, gtd, label_f, jnp.zeros((BN, 7), jnp.float32)], axis=1)

    pad = jnp.zeros((1, CP - C), jnp.float32)
    cvec = jnp.concatenate([jnp.sum(ts, axis=0, keepdims=True), pad], axis=1)
    lvec = jnp.concatenate([jnp.sum(ts * nll, axis=0, keepdims=True), pad],
                           axis=1)

    step = pl.program_id(0)

    @pl.when(step == 0)
    def _():
        cnt_ref[...] = jnp.zeros((1, CP), jnp.float32)
        lsum_ref[...] = jnp.zeros((1, CP), jnp.float32)

    cnt_ref[...] += cvec
    lsum_ref[...] += lvec


def _make_sc_kernel():
    mesh = plsc.VectorSubcoreMesh(core_axis_name="c", subcore_axis_name="s")

    @functools.partial(
        pl.kernel,
        mesh=mesh,
        out_type=jax.ShapeDtypeStruct((NW, 16), jnp.float32),
        compiler_params=pltpu.CompilerParams(
            needs_layout_passes=False, use_tc_tiling_on_sc=False),
        scratch_types=[
            pltpu.VMEM((PER_W, 16), jnp.float32),
            pltpu.VMEM((1, CP), jnp.float32),
            pltpu.VMEM((CP,), jnp.float32),
            pltpu.VMEM((PER_W,), jnp.float32),
            pltpu.VMEM((16,), jnp.float32),
        ],
    )
    def sc_kernel(gd_hbm, cnt_hbm, out_hbm, gd_v, cnt_v, wtab_v, w_all, acc_v):
        wid = lax.axis_index("c") * 16 + lax.axis_index("s")
        pltpu.sync_copy(gd_hbm.at[pl.ds(wid * PER_W, PER_W)], gd_v)
        pltpu.sync_copy(cnt_hbm, cnt_v)

        total = jnp.zeros((16,), jnp.float32)
        for k in range(CP // 16):
            total = total + cnt_v[0, pl.ds(k * 16, 16)]
        total = jnp.sum(total)
        p_fg = total - cnt_v[0, pl.ds(0, 16)][0]
        lane = lax.iota(jnp.int32, 16)
        for k in range(CP // 16):
            cv = cnt_v[0, pl.ds(k * 16, 16)]
            w = 1.0 / (1.0 + jnp.exp(-(p_fg / jnp.maximum(cv, EPS))))
            if k == 0:
                w = jnp.where(lane == 0, 0.0, w)
            wtab_v[pl.ds(k * 16, 16)] = w

        lane8 = jnp.full((16,), 8, jnp.int32)
        for k in range(PER_W // 16):
            lblf = plsc.load_gather(gd_v, [k * 16 + lane, lane8])
            lbl = lblf.astype(jnp.int32)
            w_all[pl.ds(k * 16, 16)] = plsc.load_gather(wtab_v, [lbl])

        def body(g, acc):
            e = g * 16 + lane
            row = e // 4
            jj = e % 4
            od = plsc.load_gather(gd_v, [row, jj])
            td = plsc.load_gather(gd_v, [row, jj + 4])
            w = plsc.load_gather(w_all, [row])
            d = jnp.abs(od - td) * w
            h = jnp.where(d < 1.0, 0.5 * d * d, d - 0.5)
            return acc + h

        acc = lax.fori_loop(0, PER_W * 4 // 16, body,
                            jnp.zeros((16,), jnp.float32))
        acc_v[...] = acc
        pltpu.sync_copy(acc_v, out_hbm.at[wid])

    return sc_kernel


def _final_body(nrows, cnt_ref, lsum_ref, part_ref, out_ref):
    cnt = cnt_ref[...]
    lsum = lsum_ref[...]
    part = part_ref[...]
    total = jnp.sum(cnt)
    p_fg = total - cnt[0, 0]
    w_cls = 1.0 / (1.0 + jnp.exp(-(total / jnp.maximum(cnt, EPS))))
    cls = jnp.sum(w_cls * lsum) / nrows
    reg = jnp.sum(part) / jnp.maximum(EPS, p_fg)
    out_ref[...] = jnp.broadcast_to(cls + reg, (1, 1))


def kernel(target_deltas, target_scores, output_deltas, output_scores):
    b, n, c = target_scores.shape
    nt = b * n

    gd_p, counts, lsum = pl.pallas_call(
        _stats_body,
        grid=(nt // BN,),
        in_specs=[
            pl.BlockSpec((BN, c), lambda i: (i, 0)),
            pl.BlockSpec((BN, c), lambda i: (i, 0)),
            pl.BlockSpec((BN, 4 * c), lambda i: (i, 0)),
            pl.BlockSpec((BN, 4 * c), lambda i: (i, 0)),
        ],
        out_specs=[
            pl.BlockSpec((BN, 16), lambda i: (i, 0)),
            pl.BlockSpec((1, CP), lambda i: (0, 0)),
            pl.BlockSpec((1, CP), lambda i: (0, 0)),
        ],
        out_shape=[
            jax.ShapeDtypeStruct((nt, 16), jnp.float32),
            jax.ShapeDtypeStruct((1, CP), jnp.float32),
            jax.ShapeDtypeStruct((1, CP), jnp.float32),
        ],
    )(target_scores.reshape(nt, c), output_scores.reshape(nt, c),
      output_deltas.reshape(nt, 4 * c), target_deltas.reshape(nt, 4 * c))

    npad = NW * PER_W
    gd_pad = jnp.pad(gd_p, ((0, npad - nt), (0, 0)))

    partials = _make_sc_kernel()(gd_pad, counts)

    out = pl.pallas_call(
        functools.partial(_final_body, float(nt)),
        out_shape=jax.ShapeDtypeStruct((1, 1), jnp.float32),
    )(counts, lsum, partials)
    return out[0, 0]
